# baseline (speedup 1.0000x reference)
import functools

import jax
import jax.numpy as jnp
from jax import lax
from jax.experimental import pallas as pl
from jax.experimental.pallas import tpu as pltpu
from jax.experimental.pallas import tpu_sc as plsc

N = 10000
E = 320000
D = 128
G = 64
NC, NS = 2, 16
NW = NC * NS
EPW = E // NW
K = 80
NCHUNK = EPW // K
NP = 10240
RPT = NP // NS
RCH = 32
RBLK = 1000
NBLK = N // RBLK


def _agg_body(x_hbm, src_hbm, dst_hbm, out_hbm,
              sidx0, sidx1, sidx2, sidx3, sidx4, sidx5, sidx6, sidx7,
              didx0, didx1, didx2, didx3, didx4, didx5, didx6, didx7,
              rows0, rows1, rows2, rows3, zbuf, acc,
              semi0, semi1, semi2, semi3, semi4, semi5, semi6, semi7,
              semg0, semg1, semg2, semg3,
              sems0, sems1, sems2, sems3):
    c = lax.axis_index("c")
    s = lax.axis_index("s")

    w = s * NC + c
    base = w * EPW
    sidx = (sidx0, sidx1, sidx2, sidx3, sidx4, sidx5, sidx6, sidx7)
    didx = (didx0, didx1, didx2, didx3, didx4, didx5, didx6, didx7)
    rows = (rows0, rows1, rows2, rows3)
    semi = (semi0, semi1, semi2, semi3, semi4, semi5, semi6, semi7)
    semg = (semg0, semg1, semg2, semg3)
    sems = (sems0, sems1, sems2, sems3)

    def _start_idx(i, q):
        pltpu.async_copy(src_hbm.at[pl.ds(base + i * K, K)], sidx[q], semi[q])
        pltpu.async_copy(dst_hbm.at[pl.ds(base + i * K, K)], didx[q], semi[q])

    def _wait_idx(i, q):
        pltpu.make_async_copy(src_hbm.at[pl.ds(base + i * K, K)], sidx[q], semi[q]).wait()
        pltpu.make_async_copy(dst_hbm.at[pl.ds(base + i * K, K)], didx[q], semi[q]).wait()

    def _start_gather(r, q):
        pltpu.async_copy(x_hbm.at[sidx[q]], rows[r], semg[r])

    def _wait_gather(r, q):
        pltpu.make_async_copy(x_hbm.at[sidx[q]], rows[r], semg[r]).wait()

    def _start_scatter(r, q):
        pltpu.make_async_copy(rows[r], acc.at[didx[q]], sems[r]).start(add=True)

    def _wait_scatter(r, q):
        pltpu.make_async_copy(rows[r], acc.at[didx[q]], sems[r]).wait()

    def _stage(jt, a):
        r, q = a % 4, a % 8
        r3, q3 = (a + 3) % 4, (a + 3) % 8
        q7 = (a + 7) % 8
        q5 = (a + 5) % 8

        @pl.when(jt >= 1)
        def _():
            _wait_scatter(r3, q7)

        @pl.when(jt + 3 < NCHUNK)
        def _():
            _wait_idx(jt + 3, q3)
            _start_gather(r3, q3)

        @pl.when(jt + 5 < NCHUNK)
        def _():
            _start_idx(jt + 5, q5)

        _wait_gather(r, q)
        _start_scatter(r, q)

    for i in range(5):
        _start_idx(i, i)
    for i in range(3):
        _wait_idx(i, i)
        _start_gather(i, i)

    def _zrow(i, carry):
        rr = i // 8
        jj = i - rr * 8
        zbuf[rr, pl.ds(jj * 16, 16)] = jnp.zeros((16,), jnp.float32)
        return carry

    lax.fori_loop(0, RCH * 8, _zrow, 0)
    for j in range(RPT // RCH):
        pltpu.sync_copy(zbuf, acc.at[pl.ds(s * RPT + j * RCH, RCH)])
    plsc.subcore_barrier()

    NMAIN = (NCHUNK // 8) * 8

    @pl.loop(0, NMAIN, step=8)
    def _chunks(j):
        for a in range(8):
            _stage(j + a, a)

    for jj in range(NMAIN, NCHUNK):
        _stage(jnp.int32(jj), jj % 8)

    _wait_scatter((NCHUNK - 1) % 4, (NCHUNK - 1) % 8)
    plsc.subcore_barrier()
    pltpu.sync_copy(acc.at[pl.ds(s * RPT, RPT)], out_hbm.at[c, pl.ds(s * RPT, RPT)])


@functools.cache
def _make_agg():
    return pl.kernel(
        _agg_body,
        out_type=jax.ShapeDtypeStruct((NC, NP, D), jnp.float32),
        mesh=plsc.VectorSubcoreMesh(
            core_axis_name="c", subcore_axis_name="s", num_cores=NC, num_subcores=NS
        ),
        scratch_types=(
            [pltpu.VMEM((K,), jnp.int32)] * 16
            + [pltpu.VMEM((K, D), jnp.float32)] * 4
            + [pltpu.VMEM((RCH, D), jnp.float32)]
            + [pltpu.VMEM_SHARED((NP, D), jnp.float32)]
            + [pltpu.SemaphoreType.DMA] * 16
        ),
    )


def _agg(x, src, dst):
    return _make_agg()(x, src, dst)


def _mlp_block(x_ref, p_ref, wa_ref, ba_ref, wb_ref, bb_ref, o_ref):
    hin = x_ref[...] + p_ref[0] + p_ref[1]
    t = jnp.dot(hin, wa_ref[...], preferred_element_type=jnp.float32) + ba_ref[...]
    t = jax.nn.sigmoid(t)
    h = jnp.dot(t, wb_ref[...], preferred_element_type=jnp.float32) + bb_ref[...]
    o_ref[...] = jnp.maximum(h, 0.0)


def _mlp(x, p, wa, ba, wb, bb):
    row_spec = pl.BlockSpec((RBLK, D), lambda i: (i, 0))
    full = pl.BlockSpec((D, D), lambda i: (0, 0))
    bias = pl.BlockSpec((1, D), lambda i: (0, 0))
    return pl.pallas_call(
        _mlp_block,
        grid=(NBLK,),
        in_specs=[
            row_spec,
            pl.BlockSpec((NC, RBLK, D), lambda i: (0, i, 0)),
            full, bias, full, bias,
        ],
        out_specs=row_spec,
        out_shape=jax.ShapeDtypeStruct((N, D), jnp.float32),
    )(x, p, wa, ba, wb, bb)


def _pool_block(x_ref, p_ref, wa_ref, ba_ref, wb_ref, bb_ref, batch_ref,
                wf1_ref, bf1_ref, wf2_ref, bf2_ref, o_ref, acc, cnt):
    i = pl.program_id(0)

    @pl.when(i == 0)
    def _init():
        acc[...] = jnp.zeros((G, D), jnp.float32)
        cnt[...] = jnp.zeros((G, D), jnp.float32)

    hin = x_ref[...] + p_ref[0] + p_ref[1]
    t = jnp.dot(hin, wa_ref[...], preferred_element_type=jnp.float32) + ba_ref[...]
    t = jax.nn.sigmoid(t)
    h = jnp.dot(t, wb_ref[...], preferred_element_type=jnp.float32) + bb_ref[...]
    h = jnp.maximum(h, 0.0)

    b = batch_ref[0, 0, :]
    gids = lax.broadcasted_iota(jnp.int32, (G, RBLK), 0)
    mask = (b[None, :] == gids).astype(jnp.float32)
    acc[...] += jnp.dot(mask, h, preferred_element_type=jnp.float32)
    cnt[...] += jnp.sum(mask, axis=1, keepdims=True)

    @pl.when(i == NBLK - 1)
    def _head():
        pooled = acc[...] / jnp.maximum(cnt[...], 1.0)
        o = jnp.dot(pooled, wf1_ref[...], preferred_element_type=jnp.float32) + bf1_ref[...]
        o = jax.nn.sigmoid(o)
        o_ref[...] = jnp.dot(o, wf2_ref[...], preferred_element_type=jnp.float32) + bf2_ref[...]


def _pool_head(x, p, wa, ba, wb, bb, batch3, wf1, bf1, wf2, bf2):
    row_spec = pl.BlockSpec((RBLK, D), lambda i: (i, 0))
    full = pl.BlockSpec((D, D), lambda i: (0, 0))
    bias = pl.BlockSpec((1, D), lambda i: (0, 0))
    return pl.pallas_call(
        _pool_block,
        grid=(NBLK,),
        in_specs=[
            row_spec,
            pl.BlockSpec((NC, RBLK, D), lambda i: (0, i, 0)),
            full, bias, full, bias,
            pl.BlockSpec((1, 1, RBLK), lambda i: (i, 0, 0)),
            full, bias, full, bias,
        ],
        out_specs=pl.BlockSpec((G, D), lambda i: (0, 0)),
        out_shape=jax.ShapeDtypeStruct((G, D), jnp.float32),
        scratch_shapes=[
            pltpu.VMEM((G, D), jnp.float32),
            pltpu.VMEM((G, D), jnp.float32),
        ],
    )(x, p, wa, ba, wb, bb, batch3, wf1, bf1, wf2, bf2)


def kernel(x, W1, b1, W2, b2, W3, b3, W4, b4, Wf1, bf1, Wf2, bf2, edge_index, batch):
    src = edge_index[0].astype(jnp.int32)
    dst = edge_index[1].astype(jnp.int32)
    batch3 = batch.astype(jnp.int32).reshape(NBLK, 1, RBLK)

    p1 = _agg(x, src, dst)
    h1 = _mlp(x, p1, W1, b1.reshape(1, D), W2, b2.reshape(1, D))
    p2 = _agg(h1, src, dst)
    return _pool_head(
        h1, p2, W3, b3.reshape(1, D), W4, b4.reshape(1, D), batch3,
        Wf1, bf1.reshape(1, D), Wf2, bf2.reshape(1, D),
    )


# speedup vs baseline: 1.0016x; 1.0016x over previous
import functools

import jax
import jax.numpy as jnp
from jax import lax
from jax.experimental import pallas as pl
from jax.experimental.pallas import tpu as pltpu
from jax.experimental.pallas import tpu_sc as plsc

N = 10000
E = 320000
D = 128
G = 64
NC, NS = 2, 16
NW = NC * NS
EPW = E // NW
K = 80
NCHUNK = EPW // K
NP = 10240
RPT = NP // NS
RCH = 32
RBLK = 1000
NBLK = N // RBLK


def _agg_body(x_hbm, src_hbm, dst_hbm, out_hbm,
              sidx0, sidx1, sidx2, sidx3, sidx4, sidx5, sidx6, sidx7,
              didx0, didx1, didx2, didx3, didx4, didx5, didx6, didx7,
              rows0, rows1, rows2, rows3, zbuf, acc,
              semi0, semi1, semi2, semi3, semi4, semi5, semi6, semi7,
              semg0, semg1, semg2, semg3):
    c = lax.axis_index("c")
    s = lax.axis_index("s")

    w = s * NC + c
    base = w * EPW
    sidx = (sidx0, sidx1, sidx2, sidx3, sidx4, sidx5, sidx6, sidx7)
    didx = (didx0, didx1, didx2, didx3, didx4, didx5, didx6, didx7)
    rows = (rows0, rows1, rows2, rows3)
    semi = (semi0, semi1, semi2, semi3, semi4, semi5, semi6, semi7)
    semg = (semg0, semg1, semg2, semg3)

    def _start_idx(i, q):
        pltpu.async_copy(src_hbm.at[pl.ds(base + i * K, K)], sidx[q], semi[q])
        pltpu.async_copy(dst_hbm.at[pl.ds(base + i * K, K)], didx[q], semi[q])

    def _wait_idx(i, q):
        pltpu.make_async_copy(src_hbm.at[pl.ds(base + i * K, K)], sidx[q], semi[q]).wait()
        pltpu.make_async_copy(dst_hbm.at[pl.ds(base + i * K, K)], didx[q], semi[q]).wait()

    def _start_gather(r, q):
        pltpu.async_copy(x_hbm.at[sidx[q]], rows[r], semg[r])

    def _wait_gather(r, q):
        pltpu.make_async_copy(x_hbm.at[sidx[q]], rows[r], semg[r]).wait()

    def _stage(jt, a):
        r, q = a % 4, a % 8
        r3, q3 = (a + 3) % 4, (a + 3) % 8
        q5 = (a + 5) % 8

        @pl.when(jt + 3 < NCHUNK)
        def _():
            _wait_idx(jt + 3, q3)
            _start_gather(r3, q3)

        @pl.when(jt + 5 < NCHUNK)
        def _():
            _start_idx(jt + 5, q5)

        _wait_gather(r, q)
        pltpu.sync_copy(rows[r], acc.at[didx[q]], add=True)

    for i in range(5):
        _start_idx(i, i)
    for i in range(3):
        _wait_idx(i, i)
        _start_gather(i, i)

    def _zrow(i, carry):
        rr = i // 8
        jj = i - rr * 8
        zbuf[rr, pl.ds(jj * 16, 16)] = jnp.zeros((16,), jnp.float32)
        return carry

    lax.fori_loop(0, RCH * 8, _zrow, 0)
    for j in range(RPT // RCH):
        pltpu.sync_copy(zbuf, acc.at[pl.ds(s * RPT + j * RCH, RCH)])
    plsc.subcore_barrier()

    NMAIN = (NCHUNK // 8) * 8

    @pl.loop(0, NMAIN, step=8)
    def _chunks(j):
        for a in range(8):
            _stage(j + a, a)

    for jj in range(NMAIN, NCHUNK):
        _stage(jnp.int32(jj), jj % 8)

    plsc.subcore_barrier()
    pltpu.sync_copy(acc.at[pl.ds(s * RPT, RPT)], out_hbm.at[c, pl.ds(s * RPT, RPT)])


@functools.cache
def _make_agg():
    return pl.kernel(
        _agg_body,
        out_type=jax.ShapeDtypeStruct((NC, NP, D), jnp.float32),
        mesh=plsc.VectorSubcoreMesh(
            core_axis_name="c", subcore_axis_name="s", num_cores=NC, num_subcores=NS
        ),
        scratch_types=(
            [pltpu.VMEM((K,), jnp.int32)] * 16
            + [pltpu.VMEM((K, D), jnp.float32)] * 4
            + [pltpu.VMEM((RCH, D), jnp.float32)]
            + [pltpu.VMEM_SHARED((NP, D), jnp.float32)]
            + [pltpu.SemaphoreType.DMA] * 12
        ),
    )


def _agg(x, src, dst):
    return _make_agg()(x, src, dst)


def _mlp_block(x_ref, p_ref, wa_ref, ba_ref, wb_ref, bb_ref, o_ref):
    hin = x_ref[...] + p_ref[0] + p_ref[1]
    t = jnp.dot(hin, wa_ref[...], preferred_element_type=jnp.float32) + ba_ref[...]
    t = jax.nn.sigmoid(t)
    h = jnp.dot(t, wb_ref[...], preferred_element_type=jnp.float32) + bb_ref[...]
    o_ref[...] = jnp.maximum(h, 0.0)


def _mlp(x, p, wa, ba, wb, bb):
    row_spec = pl.BlockSpec((RBLK, D), lambda i: (i, 0))
    full = pl.BlockSpec((D, D), lambda i: (0, 0))
    bias = pl.BlockSpec((1, D), lambda i: (0, 0))
    return pl.pallas_call(
        _mlp_block,
        grid=(NBLK,),
        in_specs=[
            row_spec,
            pl.BlockSpec((NC, RBLK, D), lambda i: (0, i, 0)),
            full, bias, full, bias,
        ],
        out_specs=row_spec,
        out_shape=jax.ShapeDtypeStruct((N, D), jnp.float32),
    )(x, p, wa, ba, wb, bb)


def _pool_block(x_ref, p_ref, wa_ref, ba_ref, wb_ref, bb_ref, batch_ref,
                wf1_ref, bf1_ref, wf2_ref, bf2_ref, o_ref, acc, cnt):
    i = pl.program_id(0)

    @pl.when(i == 0)
    def _init():
        acc[...] = jnp.zeros((G, D), jnp.float32)
        cnt[...] = jnp.zeros((G, D), jnp.float32)

    hin = x_ref[...] + p_ref[0] + p_ref[1]
    t = jnp.dot(hin, wa_ref[...], preferred_element_type=jnp.float32) + ba_ref[...]
    t = jax.nn.sigmoid(t)
    h = jnp.dot(t, wb_ref[...], preferred_element_type=jnp.float32) + bb_ref[...]
    h = jnp.maximum(h, 0.0)

    b = batch_ref[0, 0, :]
    gids = lax.broadcasted_iota(jnp.int32, (G, RBLK), 0)
    mask = (b[None, :] == gids).astype(jnp.float32)
    acc[...] += jnp.dot(mask, h, preferred_element_type=jnp.float32)
    cnt[...] += jnp.sum(mask, axis=1, keepdims=True)

    @pl.when(i == NBLK - 1)
    def _head():
        pooled = acc[...] / jnp.maximum(cnt[...], 1.0)
        o = jnp.dot(pooled, wf1_ref[...], preferred_element_type=jnp.float32) + bf1_ref[...]
        o = jax.nn.sigmoid(o)
        o_ref[...] = jnp.dot(o, wf2_ref[...], preferred_element_type=jnp.float32) + bf2_ref[...]


def _pool_head(x, p, wa, ba, wb, bb, batch3, wf1, bf1, wf2, bf2):
    row_spec = pl.BlockSpec((RBLK, D), lambda i: (i, 0))
    full = pl.BlockSpec((D, D), lambda i: (0, 0))
    bias = pl.BlockSpec((1, D), lambda i: (0, 0))
    return pl.pallas_call(
        _pool_block,
        grid=(NBLK,),
        in_specs=[
            row_spec,
            pl.BlockSpec((NC, RBLK, D), lambda i: (0, i, 0)),
            full, bias, full, bias,
            pl.BlockSpec((1, 1, RBLK), lambda i: (i, 0, 0)),
            full, bias, full, bias,
        ],
        out_specs=pl.BlockSpec((G, D), lambda i: (0, 0)),
        out_shape=jax.ShapeDtypeStruct((G, D), jnp.float32),
        scratch_shapes=[
            pltpu.VMEM((G, D), jnp.float32),
            pltpu.VMEM((G, D), jnp.float32),
        ],
    )(x, p, wa, ba, wb, bb, batch3, wf1, bf1, wf2, bf2)


def kernel(x, W1, b1, W2, b2, W3, b3, W4, b4, Wf1, bf1, Wf2, bf2, edge_index, batch):
    src = edge_index[0].astype(jnp.int32)
    dst = edge_index[1].astype(jnp.int32)
    batch3 = batch.astype(jnp.int32).reshape(NBLK, 1, RBLK)

    p1 = _agg(x, src, dst)
    h1 = _mlp(x, p1, W1, b1.reshape(1, D), W2, b2.reshape(1, D))
    p2 = _agg(h1, src, dst)
    return _pool_head(
        h1, p2, W3, b3.reshape(1, D), W4, b4.reshape(1, D), batch3,
        Wf1, bf1.reshape(1, D), Wf2, bf2.reshape(1, D),
    )
